# Initial kernel scaffold; baseline (speedup 1.0000x reference)
import jax
import jax.numpy as jnp
from jax.experimental import pallas as pl


def kernel(h, coord, edge_index, edge_mask, W1, b1, W2, b2, Win, Wout, bout):
    raise NotImplementedError("write your pallas kernel here")


# speedup vs baseline: 2.4136x; 2.4136x over previous
import functools

import numpy as np
import jax
import jax.numpy as jnp
from jax import lax
from jax.experimental import pallas as pl
from jax.experimental.pallas import tpu as pltpu
from jax.experimental.pallas import tpu_sc as plsc

N_NODES = 10000
N_EDGES = 160000
IN_CH = 256
OUT_CH = 256
N_FILTERS = 256
N_GAUSS = 64
CUTOFF = 10.0
HALF = N_FILTERS // 2

_centers = np.linspace(0.0, CUTOFF, N_GAUSS).astype(np.float32)
_SPACING = np.float32(_centers[1] - _centers[0])
_width = np.float32(abs(_centers[1] - _centers[0]))
_coeff = np.float32(-0.5 / (_width * _width))
_LOG2 = np.float32(np.log(2.0))

NC, NS = 2, 16
K = 128
NCHUNK = N_EDGES // K
ROWS_PER_SUB = N_NODES // NS


def _hw_body(h_ref, win_ref, out_ref):
    hblk = h_ref[...]
    w = win_ref[...]
    out_ref[0] = jnp.dot(hblk, w[:, :HALF], preferred_element_type=jnp.float32)
    out_ref[1] = jnp.dot(hblk, w[:, HALF:], preferred_element_type=jnp.float32)


def _hw_call(h, Win):
    BN = 1000
    return pl.pallas_call(
        _hw_body,
        grid=(N_NODES // BN,),
        in_specs=[
            pl.BlockSpec((BN, IN_CH), lambda i: (i, 0)),
            pl.BlockSpec((IN_CH, N_FILTERS), lambda i: (0, 0)),
        ],
        out_specs=pl.BlockSpec((2, BN, HALF), lambda i: (0, i, 0)),
        out_shape=jax.ShapeDtypeStruct((2, N_NODES, HALF), jnp.float32),
    )(h, Win)


def _gather_body(hw, coordf, row, col, ghw, d2out,
                 idx_v, rows_v, idxr_v, d2_v, tab_v, sem):
    c = lax.axis_index("c")
    s = lax.axis_index("s")
    w = s * NC + c

    pltpu.sync_copy(coordf, tab_v)

    def hw_chunk(t, carry):
        j = s + NS * t

        @pl.when(j < NCHUNK)
        def _():
            base = j * K
            pltpu.sync_copy(col.at[pl.ds(base, K)], idx_v)
            pltpu.async_copy(hw.at[c].at[idx_v], rows_v, sem).wait()
            pltpu.sync_copy(rows_v, ghw.at[c].at[pl.ds(base, K)])

        return carry

    lax.fori_loop(0, (NCHUNK + NS - 1) // NS, hw_chunk, 0)

    def d2_chunk(t, carry):
        j = w + NC * NS * t

        @pl.when(j < NCHUNK)
        def _():
            base = j * K
            pltpu.sync_copy(row.at[pl.ds(base, K)], idxr_v)
            pltpu.sync_copy(col.at[pl.ds(base, K)], idx_v)

            def sub(k, cc):
                br = idxr_v[pl.ds(k * 16, 16)] * 4
                bc = idx_v[pl.ds(k * 16, 16)] * 4
                dx = (plsc.load_gather(tab_v, [br])
                      - plsc.load_gather(tab_v, [bc]))
                dy = (plsc.load_gather(tab_v, [br + 1])
                      - plsc.load_gather(tab_v, [bc + 1]))
                dz = (plsc.load_gather(tab_v, [br + 2])
                      - plsc.load_gather(tab_v, [bc + 2]))
                d2_v[pl.ds(k * 16, 16)] = dx * dx + dy * dy + dz * dz
                return cc

            lax.fori_loop(0, K // 16, sub, 0)
            pltpu.sync_copy(d2_v, d2out.at[pl.ds(base, K)])

        return carry

    lax.fori_loop(0, (NCHUNK + NC * NS - 1) // (NC * NS), d2_chunk, 0)


_gather_call = pl.kernel(
    _gather_body,
    out_type=(
        jax.ShapeDtypeStruct((2, N_EDGES, HALF), jnp.float32),
        jax.ShapeDtypeStruct((N_EDGES,), jnp.float32),
    ),
    mesh=plsc.VectorSubcoreMesh(core_axis_name="c", subcore_axis_name="s"),
    scratch_types=[
        pltpu.VMEM((K,), jnp.int32),
        pltpu.VMEM((K, HALF), jnp.float32),
        pltpu.VMEM((K,), jnp.int32),
        pltpu.VMEM((K,), jnp.float32),
        pltpu.VMEM((4 * N_NODES,), jnp.float32),
        pltpu.SemaphoreType.DMA,
    ],
    compiler_params=pltpu.CompilerParams(needs_layout_passes=False),
)


def _filter_body(d2_ref, mask_ref, ghw_ref, w1_ref, b1_ref,
                 w2_ref, b2_ref, out_ref):
    d2 = d2_ref[...]
    d = jnp.minimum(jnp.sqrt(d2), CUTOFF)
    centers = _SPACING * lax.broadcasted_iota(
        jnp.int32, (1, N_GAUSS), 1).astype(jnp.float32)
    delta = d - centers
    rbf = jnp.exp(_coeff * (delta * delta))
    t = jnp.dot(rbf, w1_ref[...], preferred_element_type=jnp.float32)
    t = t + b1_ref[...][None, :]
    ssp = jnp.maximum(t, 0.0) + jnp.log1p(jnp.exp(-jnp.abs(t))) - _LOG2
    wf = jnp.dot(ssp, w2_ref[...], preferred_element_type=jnp.float32)
    wf = (wf + b2_ref[...][None, :]) * mask_ref[...]
    out_ref[0] = ghw_ref[0] * wf[:, :HALF]
    out_ref[1] = ghw_ref[1] * wf[:, HALF:]


def _filter_call(d2, mask, ghw, W1, b1, W2, b2):
    BE = 800
    return pl.pallas_call(
        _filter_body,
        grid=(N_EDGES // BE,),
        in_specs=[
            pl.BlockSpec((BE, 1), lambda i: (i, 0)),
            pl.BlockSpec((BE, 1), lambda i: (i, 0)),
            pl.BlockSpec((2, BE, HALF), lambda i: (0, i, 0)),
            pl.BlockSpec((N_GAUSS, N_FILTERS), lambda i: (0, 0)),
            pl.BlockSpec((N_FILTERS,), lambda i: (0,)),
            pl.BlockSpec((N_FILTERS, N_FILTERS), lambda i: (0, 0)),
            pl.BlockSpec((N_FILTERS,), lambda i: (0,)),
        ],
        out_specs=pl.BlockSpec((2, BE, HALF), lambda i: (0, i, 0)),
        out_shape=jax.ShapeDtypeStruct((2, N_EDGES, HALF), jnp.float32),
    )(d2, mask, ghw, W1, b1, W2, b2)


def _scatter_body(msg, row, agg, idx_v, msg_v, z_v, agg_sh, sem):
    c = lax.axis_index("c")
    s = lax.axis_index("s")

    def zstore(i, carry):
        r = i // (HALF // 16)
        kk = (i % (HALF // 16)) * 16
        z_v[r, pl.ds(kk, 16)] = jnp.zeros((16,), jnp.float32)
        return carry

    lax.fori_loop(0, 16 * (HALF // 16), zstore, 0)

    base0 = s * 624
    nrows = jnp.where(s == NS - 1, 640, 624)

    def zcopy(i, carry):
        pltpu.sync_copy(z_v, agg_sh.at[pl.ds(base0 + i * 16, 16)])
        return carry

    lax.fori_loop(0, nrows // 16, zcopy, 0)

    plsc.subcore_barrier()

    def schunk(t, carry):
        j = s + NS * t

        @pl.when(j < NCHUNK)
        def _():
            base = j * K
            pltpu.sync_copy(row.at[pl.ds(base, K)], idx_v)
            pltpu.sync_copy(msg.at[c].at[pl.ds(base, K)], msg_v)
            pltpu.sync_copy(msg_v, agg_sh.at[idx_v], add=True)

        return carry

    lax.fori_loop(0, (NCHUNK + NS - 1) // NS, schunk, 0)

    plsc.subcore_barrier()

    @pl.when(s == NS - 1)
    def _():
        pltpu.sync_copy(agg_sh.at[pl.ds(base0, 640)],
                        agg.at[c].at[pl.ds(base0, 640)])

    @pl.when(s < NS - 1)
    def _():
        pltpu.sync_copy(agg_sh.at[pl.ds(base0, 624)],
                        agg.at[c].at[pl.ds(base0, 624)])


_scatter_call = pl.kernel(
    _scatter_body,
    out_type=jax.ShapeDtypeStruct((2, N_NODES, HALF), jnp.float32),
    mesh=plsc.VectorSubcoreMesh(core_axis_name="c", subcore_axis_name="s"),
    scratch_types=[
        pltpu.VMEM((K,), jnp.int32),
        pltpu.VMEM((K, HALF), jnp.float32),
        pltpu.VMEM((16, HALF), jnp.float32),
        pltpu.VMEM_SHARED((N_NODES, HALF), jnp.float32),
        pltpu.SemaphoreType.DMA,
    ],
)


def _out_body(agg_ref, wout_ref, bout_ref, out_ref):
    w = wout_ref[...]
    acc = jnp.dot(agg_ref[0], w[:HALF, :], preferred_element_type=jnp.float32)
    acc = acc + jnp.dot(agg_ref[1], w[HALF:, :],
                        preferred_element_type=jnp.float32)
    out_ref[...] = acc + bout_ref[...][None, :]


def _out_call(agg, Wout, bout):
    BN = 1000
    return pl.pallas_call(
        _out_body,
        grid=(N_NODES // BN,),
        in_specs=[
            pl.BlockSpec((2, BN, HALF), lambda i: (0, i, 0)),
            pl.BlockSpec((N_FILTERS, OUT_CH), lambda i: (0, 0)),
            pl.BlockSpec((OUT_CH,), lambda i: (0,)),
        ],
        out_specs=pl.BlockSpec((BN, OUT_CH), lambda i: (i, 0)),
        out_shape=jax.ShapeDtypeStruct((N_NODES, OUT_CH), jnp.float32),
    )(agg, Wout, bout)


def kernel(h, coord, edge_index, edge_mask, W1, b1, W2, b2, Win, Wout, bout):
    row = edge_index[0].astype(jnp.int32)
    col = edge_index[1].astype(jnp.int32)
    coordf = jnp.concatenate(
        [coord, jnp.zeros((N_NODES, 1), coord.dtype)], axis=1).reshape(-1)

    hw = _hw_call(h, Win)
    ghw, d2 = _gather_call(hw, coordf, row, col)
    msg = _filter_call(d2.reshape(N_EDGES, 1), edge_mask, ghw, W1, b1, W2, b2)
    agg = _scatter_call(msg, row)
    return _out_call(agg, Wout, bout)


# speedup vs baseline: 3.2254x; 1.3363x over previous
import functools

import numpy as np
import jax
import jax.numpy as jnp
from jax import lax
from jax.experimental import pallas as pl
from jax.experimental.pallas import tpu as pltpu
from jax.experimental.pallas import tpu_sc as plsc

N_NODES = 10000
N_EDGES = 160000
IN_CH = 256
OUT_CH = 256
N_FILTERS = 256
N_GAUSS = 64
CUTOFF = 10.0
HALF = N_FILTERS // 2

_centers = np.linspace(0.0, CUTOFF, N_GAUSS).astype(np.float32)
_SPACING = np.float32(_centers[1] - _centers[0])
_coeff = np.float32(-0.5 / (_SPACING * _SPACING))
_LOG2 = np.float32(np.log(2.0))
_LN2 = np.float32(np.log(2.0))

NC, NS = 2, 16
KD = 128
NCHUNK_D = N_EDGES // KD
K2 = 80
NCHUNK_M = N_EDGES // K2
T_M = NCHUNK_M // NS


def _hw_body(h_ref, win_ref, out_ref):
    hblk = h_ref[...]
    w = win_ref[...]
    out_ref[0] = jnp.dot(hblk, w[:, :HALF], preferred_element_type=jnp.float32)
    out_ref[1] = jnp.dot(hblk, w[:, HALF:], preferred_element_type=jnp.float32)


def _hw_call(h, Win):
    BN = 1000
    return pl.pallas_call(
        _hw_body,
        grid=(N_NODES // BN,),
        in_specs=[
            pl.BlockSpec((BN, IN_CH), lambda i: (i, 0)),
            pl.BlockSpec((IN_CH, N_FILTERS), lambda i: (0, 0)),
        ],
        out_specs=pl.BlockSpec((2, BN, HALF), lambda i: (0, i, 0)),
        out_shape=jax.ShapeDtypeStruct((2, N_NODES, HALF), jnp.float32),
    )(h, Win)


def _d2_body(coordf, row, col, d2out, idxc_v, idxr_v, d2_v, tab_v):
    c = lax.axis_index("c")
    s = lax.axis_index("s")
    w = s * NC + c

    pltpu.sync_copy(coordf, tab_v)

    def d2_chunk(t, carry):
        j = w + NC * NS * t

        @pl.when(j < NCHUNK_D)
        def _():
            base = j * KD
            pltpu.sync_copy(row.at[pl.ds(base, KD)], idxr_v)
            pltpu.sync_copy(col.at[pl.ds(base, KD)], idxc_v)

            def sub(k, cc):
                br = idxr_v[pl.ds(k * 16, 16)] * 4
                bc = idxc_v[pl.ds(k * 16, 16)] * 4
                dx = (plsc.load_gather(tab_v, [br])
                      - plsc.load_gather(tab_v, [bc]))
                dy = (plsc.load_gather(tab_v, [br + 1])
                      - plsc.load_gather(tab_v, [bc + 1]))
                dz = (plsc.load_gather(tab_v, [br + 2])
                      - plsc.load_gather(tab_v, [bc + 2]))
                d2_v[pl.ds(k * 16, 16)] = dx * dx + dy * dy + dz * dz
                return cc

            lax.fori_loop(0, KD // 16, sub, 0)
            pltpu.sync_copy(d2_v, d2out.at[pl.ds(base, KD)])

        return carry

    lax.fori_loop(0, (NCHUNK_D + NC * NS - 1) // (NC * NS), d2_chunk, 0)


_d2_call = pl.kernel(
    _d2_body,
    out_type=jax.ShapeDtypeStruct((N_EDGES,), jnp.float32),
    mesh=plsc.VectorSubcoreMesh(core_axis_name="c", subcore_axis_name="s"),
    scratch_types=[
        pltpu.VMEM((KD,), jnp.int32),
        pltpu.VMEM((KD,), jnp.int32),
        pltpu.VMEM((KD,), jnp.float32),
        pltpu.VMEM((4 * N_NODES,), jnp.float32),
    ],
    compiler_params=pltpu.CompilerParams(needs_layout_passes=False),
)


def _filter_body(d2_ref, mask_ref, w1_ref, b1_ref, w2_ref, b2_ref, out_ref):
    d2 = d2_ref[...]
    d = jnp.minimum(jnp.sqrt(d2), CUTOFF)
    centers = _SPACING * lax.broadcasted_iota(
        jnp.int32, (1, N_GAUSS), 1).astype(jnp.float32)
    delta = d - centers
    rbf = jnp.exp(_coeff * (delta * delta))
    t = jnp.dot(rbf, w1_ref[...], preferred_element_type=jnp.float32)
    t = t + b1_ref[...][None, :]
    ssp = jnp.log(1.0 + jnp.exp(t)) - _LOG2
    wf = jnp.dot(ssp, w2_ref[...], preferred_element_type=jnp.float32)
    wf = (wf + b2_ref[...][None, :]) * mask_ref[...]
    out_ref[0] = wf[:, :HALF]
    out_ref[1] = wf[:, HALF:]


def _filter_call(d2, mask, W1, b1, W2, b2):
    BE = 800
    return pl.pallas_call(
        _filter_body,
        grid=(N_EDGES // BE,),
        in_specs=[
            pl.BlockSpec((BE, 1), lambda i: (i, 0)),
            pl.BlockSpec((BE, 1), lambda i: (i, 0)),
            pl.BlockSpec((N_GAUSS, N_FILTERS), lambda i: (0, 0)),
            pl.BlockSpec((N_FILTERS,), lambda i: (0,)),
            pl.BlockSpec((N_FILTERS, N_FILTERS), lambda i: (0, 0)),
            pl.BlockSpec((N_FILTERS,), lambda i: (0,)),
        ],
        out_specs=pl.BlockSpec((2, BE, HALF), lambda i: (0, i, 0)),
        out_shape=jax.ShapeDtypeStruct((2, N_EDGES, HALF), jnp.float32),
    )(d2, mask, W1, b1, W2, b2)


def _msg_body(hw, wmat, row, col, agg,
              idxc0, idxc1, idxr0, idxr1, a0, a1, w0, w1, z_v,
              agg_sh, semi0, semi1, semo0, semo1):
    c = lax.axis_index("c")
    s = lax.axis_index("s")
    idxc = (idxc0, idxc1)
    idxr = (idxr0, idxr1)
    av = (a0, a1)
    wv = (w0, w1)
    semi = (semi0, semi1)
    semo = (semo0, semo1)

    def zstore(i, carry):
        r = i // (HALF // 16)
        kk = (i % (HALF // 16)) * 16
        z_v[r, pl.ds(kk, 16)] = jnp.zeros((16,), jnp.float32)
        return carry

    lax.fori_loop(0, 16 * (HALF // 16), zstore, 0)

    base0 = s * 624
    nrows = jnp.where(s == NS - 1, 640, 624)

    def zcopy(i, carry):
        pltpu.sync_copy(z_v, agg_sh.at[pl.ds(base0 + i * 16, 16)])
        return carry

    lax.fori_loop(0, nrows // 16, zcopy, 0)
    plsc.subcore_barrier()

    def issue(t, b):
        @pl.when(t < T_M)
        def _():
            @pl.when(t >= 2)
            def _():
                pltpu.make_async_copy(
                    av[b], agg_sh.at[idxr[b]], semo[b]).wait()

            ch = s + NS * t
            base = ch * K2
            pltpu.sync_copy(col.at[pl.ds(base, K2)], idxc[b])
            pltpu.sync_copy(row.at[pl.ds(base, K2)], idxr[b])
            pltpu.async_copy(hw.at[c].at[idxc[b]], av[b], semi[b])
            pltpu.async_copy(wmat.at[c].at[pl.ds(base, K2)], wv[b], semi[b])

    def compute(t, b):
        @pl.when(t < T_M)
        def _():
            ch = s + NS * t
            base = ch * K2
            pltpu.make_async_copy(hw.at[c].at[idxc[b]], av[b], semi[b]).wait()
            pltpu.make_async_copy(
                wmat.at[c].at[pl.ds(base, K2)], wv[b], semi[b]).wait()

            def mrow(r, cc):
                for kk in range(HALF // 16):
                    sl = pl.ds(kk * 16, 16)
                    av[b][r, sl] = av[b][r, sl] * wv[b][r, sl]
                return cc

            lax.fori_loop(0, K2, mrow, 0)
            pltpu.async_copy(av[b], agg_sh.at[idxr[b]], semo[b], add=True)

    issue(0, 0)

    def body(tt, carry):
        t0 = 2 * tt
        issue(t0 + 1, 1)
        compute(t0, 0)
        issue(t0 + 2, 0)
        compute(t0 + 1, 1)
        return carry

    lax.fori_loop(0, (T_M + 1) // 2, body, 0)

    pltpu.make_async_copy(av[0], agg_sh.at[idxr[0]], semo[0]).wait()
    pltpu.make_async_copy(av[1], agg_sh.at[idxr[1]], semo[1]).wait()

    plsc.subcore_barrier()

    @pl.when(s == NS - 1)
    def _():
        pltpu.sync_copy(agg_sh.at[pl.ds(base0, 640)],
                        agg.at[c].at[pl.ds(base0, 640)])

    @pl.when(s < NS - 1)
    def _():
        pltpu.sync_copy(agg_sh.at[pl.ds(base0, 624)],
                        agg.at[c].at[pl.ds(base0, 624)])


_msg_call = pl.kernel(
    _msg_body,
    out_type=jax.ShapeDtypeStruct((2, N_NODES, HALF), jnp.float32),
    mesh=plsc.VectorSubcoreMesh(core_axis_name="c", subcore_axis_name="s"),
    scratch_types=[
        pltpu.VMEM((K2,), jnp.int32),
        pltpu.VMEM((K2,), jnp.int32),
        pltpu.VMEM((K2,), jnp.int32),
        pltpu.VMEM((K2,), jnp.int32),
        pltpu.VMEM((K2, HALF), jnp.float32),
        pltpu.VMEM((K2, HALF), jnp.float32),
        pltpu.VMEM((K2, HALF), jnp.float32),
        pltpu.VMEM((K2, HALF), jnp.float32),
        pltpu.VMEM((16, HALF), jnp.float32),
        pltpu.VMEM_SHARED((N_NODES, HALF), jnp.float32),
        pltpu.SemaphoreType.DMA,
        pltpu.SemaphoreType.DMA,
        pltpu.SemaphoreType.DMA,
        pltpu.SemaphoreType.DMA,
    ],
    compiler_params=pltpu.CompilerParams(needs_layout_passes=False),
)


def _out_body(agg_ref, wout_ref, bout_ref, out_ref):
    w = wout_ref[...]
    acc = jnp.dot(agg_ref[0], w[:HALF, :], preferred_element_type=jnp.float32)
    acc = acc + jnp.dot(agg_ref[1], w[HALF:, :],
                        preferred_element_type=jnp.float32)
    out_ref[...] = acc + bout_ref[...][None, :]


def _out_call(agg, Wout, bout):
    BN = 1000
    return pl.pallas_call(
        _out_body,
        grid=(N_NODES // BN,),
        in_specs=[
            pl.BlockSpec((2, BN, HALF), lambda i: (0, i, 0)),
            pl.BlockSpec((N_FILTERS, OUT_CH), lambda i: (0, 0)),
            pl.BlockSpec((OUT_CH,), lambda i: (0,)),
        ],
        out_specs=pl.BlockSpec((BN, OUT_CH), lambda i: (i, 0)),
        out_shape=jax.ShapeDtypeStruct((N_NODES, OUT_CH), jnp.float32),
    )(agg, Wout, bout)


def kernel(h, coord, edge_index, edge_mask, W1, b1, W2, b2, Win, Wout, bout):
    row = edge_index[0].astype(jnp.int32)
    col = edge_index[1].astype(jnp.int32)
    coordf = jnp.concatenate(
        [coord, jnp.zeros((N_NODES, 1), coord.dtype)], axis=1).reshape(-1)

    hw = _hw_call(h, Win)
    d2 = _d2_call(coordf, row, col)
    wmat = _filter_call(d2.reshape(N_EDGES, 1), edge_mask, W1, b1, W2, b2)
    agg = _msg_call(hw, wmat, row, col)
    return _out_call(agg, Wout, bout)
